# Initial kernel scaffold; baseline (speedup 1.0000x reference)
import jax
import jax.numpy as jnp
from jax.experimental import pallas as pl


def kernel(x, c_0, c_1, c_2, c_3, c_4):
    raise NotImplementedError("write your pallas kernel here")


# speedup vs baseline: 168.0286x; 168.0286x over previous
import functools

import jax
import jax.numpy as jnp
from jax import lax
from jax.experimental import pallas as pl
from jax.experimental.pallas import tpu as pltpu
from jax.experimental.pallas import tpu_sc as plsc

_NUM_CORES = 2
_NUM_SUBCORES = 16
_NUM_WORKERS = _NUM_CORES * _NUM_SUBCORES
_LANES = 16
_CHUNK_ROWS = 16


@functools.cache
def _build_scatter(n_rows: int, n_cols: int, nnz_per_row: int):
    rows_per_worker = n_rows // _NUM_WORKERS
    chunks_per_worker = rows_per_worker // _CHUNK_ROWS
    chunk_nnz = _CHUNK_ROWS * nnz_per_row
    stage_words = ((chunk_nnz + 127) // 128) * 128

    mesh = plsc.VectorSubcoreMesh(
        core_axis_name="c", subcore_axis_name="s",
        num_cores=_NUM_CORES, num_subcores=_NUM_SUBCORES)

    @functools.partial(
        pl.kernel,
        out_type=jax.ShapeDtypeStruct((n_rows, n_cols), jnp.float32),
        mesh=mesh,
        compiler_params=pltpu.CompilerParams(needs_layout_passes=False),
        scratch_types=[
            pltpu.VMEM((stage_words,), jnp.float32),
            pltpu.VMEM((stage_words,), jnp.int32),
            pltpu.VMEM((_CHUNK_ROWS, n_cols), jnp.float32),
        ],
    )
    def scatter_kernel(c0_hbm, c1_hbm, w_hbm, vals_v, cols_v, buf):
        wid = lax.axis_index("s") * _NUM_CORES + lax.axis_index("c")
        row_base = wid * rows_per_worker

        iot = lax.iota(jnp.int32, _LANES)
        zz = jnp.zeros((_LANES,), jnp.float32)

        def _zero(i, carry):
            for r in range(_CHUNK_ROWS):
                buf[r, pl.ds(i * _LANES, _LANES)] = zz
            return carry
        lax.fori_loop(0, n_cols // _LANES, _zero, 0)

        gbase = iot * nnz_per_row

        def _chunk(c, carry):
            r0 = row_base + c * _CHUNK_ROWS
            off = r0 * nnz_per_row
            pltpu.sync_copy(c0_hbm.at[pl.ds(off, chunk_nnz)],
                            vals_v.at[pl.ds(0, chunk_nnz)])
            pltpu.sync_copy(c1_hbm.at[pl.ds(off, chunk_nnz)],
                            cols_v.at[pl.ds(0, chunk_nnz)])
            for k in range(nnz_per_row):
                colk = plsc.load_gather(cols_v, [gbase + k])
                valk = plsc.load_gather(vals_v, [gbase + k])
                plsc.addupdate_scatter(buf, [iot, colk], valk)
            pltpu.sync_copy(buf, w_hbm.at[pl.ds(r0, _CHUNK_ROWS)])
            for k in range(nnz_per_row):
                colk = plsc.load_gather(cols_v, [gbase + k])
                plsc.store_scatter(buf, [iot, colk], zz)
            return carry
        lax.fori_loop(0, chunks_per_worker, _chunk, 0)

    return scatter_kernel


@functools.cache
def _build_matmul(m: int, k: int, n: int):
    bn = 512

    def mm_body(x_ref, w_ref, o_ref):
        o_ref[...] = lax.dot_general(
            x_ref[...], w_ref[...],
            dimension_numbers=(((1,), (1,)), ((), ())),
            preferred_element_type=jnp.float32)

    return pl.pallas_call(
        mm_body,
        grid=(n // bn,),
        in_specs=[
            pl.BlockSpec((m, k), lambda i: (0, 0)),
            pl.BlockSpec((bn, k), lambda i: (i, 0)),
        ],
        out_specs=pl.BlockSpec((m, bn), lambda i: (0, i)),
        out_shape=jax.ShapeDtypeStruct((m, n), jnp.float32),
    )


def kernel(x, c_0, c_1, c_2, c_3, c_4):
    original_shape = x.shape
    n_cols = original_shape[-1]
    n_rows = c_2.shape[0] - 1
    nnz_per_row = c_0.shape[0] // n_rows
    x_flat = x.reshape(-1, n_cols)

    w = _build_scatter(n_rows, n_cols, nnz_per_row)(c_0, c_1)
    out_flat = _build_matmul(x_flat.shape[0], n_cols, n_rows)(x_flat, w)
    return out_flat.reshape(*original_shape[:-1], n_rows)
